# Initial kernel scaffold; baseline (speedup 1.0000x reference)
import jax
import jax.numpy as jnp
from jax.experimental import pallas as pl


def kernel(struct, seq, token_to_decode, W_e, W_s, W1, W2, W_out):
    raise NotImplementedError("write your pallas kernel here")


# speedup vs baseline: 10.3807x; 10.3807x over previous
import functools

import jax
import jax.numpy as jnp
from jax import lax
from jax.experimental import pallas as pl
from jax.experimental.pallas import tpu as pltpu
from jax.experimental.pallas import tpu_sc as plsc

N, L, K, D = 8, 1024, 48, 128
NL = N * L
NLK = NL * K
LBLK = 256
PBLK = 512
CBLK = 128
NWORKERS = 32
CH = 128

_dot = functools.partial(
    jnp.dot, precision=lax.Precision.HIGHEST, preferred_element_type=jnp.float32
)


def _knn_body(ca_i_ref, caT_ref, seq_ref, ws_ref, idx_ref, rbf_ref, h0_ref):
    b = pl.program_id(0)
    n_off = (b // (L // LBLK)) * L
    d2 = jnp.zeros((LBLK, L), jnp.float32)
    for c in range(3):
        xi = ca_i_ref[:, c : c + 1]
        xj = caT_ref[0, c : c + 1, :]
        diff = xi - xj
        d2 = d2 + diff * diff
    jidx = lax.broadcasted_iota(jnp.int32, (LBLK, L), 1)
    vals = d2
    idx_cols = []
    d_cols = []
    for _ in range(K):
        m = jnp.min(vals, axis=1, keepdims=True)
        am = jnp.min(
            jnp.where(vals == m, jidx, jnp.int32(1 << 30)), axis=1, keepdims=True
        )
        idx_cols.append(am)
        d_cols.append(m)
        vals = jnp.where(jidx == am, jnp.float32(3.0e38), vals)
    idxs = jnp.concatenate(idx_cols, axis=1)
    dsel = jnp.sqrt(jnp.concatenate(d_cols, axis=1) + 1e-6)
    idx_ref[...] = idxs + n_off
    mu = 2.0 + (20.0 / 15.0) * lax.broadcasted_iota(
        jnp.int32, (LBLK, K, 16), 2
    ).astype(jnp.float32)
    z = (dsel[:, :, None] - mu) * (1.0 / 1.25)
    rbf_ref[...] = jnp.exp(-(z * z))
    s = seq_ref[...]
    oh = jnp.where(
        s == lax.broadcasted_iota(jnp.int32, (LBLK, 32), 1),
        jnp.float32(1.0),
        jnp.float32(0.0),
    )
    h0_ref[...] = _dot(oh, ws_ref[...])


def _proj_body(h_ref, w_ref, p_ref, g_ref):
    pg = _dot(h_ref[...], w_ref[...])
    p_ref[...] = pg[:, :D]
    g_ref[...] = pg[:, D:]


def _combine_body(gj_ref, rbf_ref, p_ref, h_ref, we_ref, w1c_ref, w2_ref, out_ref):
    u = _dot(we_ref[...], w1c_ref[...])
    r = rbf_ref[...].reshape(CBLK * K, 16)
    f = _dot(r, u)
    a = (gj_ref[...] + f).reshape(CBLK, K, D) + p_ref[...][:, None, :]
    s = jnp.maximum(a, 0.0).sum(axis=1) * (1.0 / K)
    x = h_ref[...] + _dot(s, w2_ref[...])
    m = jnp.mean(x, axis=1, keepdims=True)
    xc = x - m
    var = jnp.mean(xc * xc, axis=1, keepdims=True)
    out_ref[...] = xc / jnp.sqrt(var + 1e-5)


def _out_body(tok_ref, h_ref, wout_ref, out_ref):
    n = pl.program_id(0)
    row = h_ref[0, pl.ds(tok_ref[n], 1), :]
    logits = _dot(row, wout_ref[...])
    lane = lax.broadcasted_iota(jnp.int32, (1, D), 1)
    logits = jnp.where(lane < 20, logits, jnp.float32(-3.0e38))
    mx = jnp.max(logits, axis=1, keepdims=True)
    e = jnp.exp(logits - mx)
    out_ref[0] = e / jnp.sum(e, axis=1, keepdims=True)


def _sc_gather(idx_flat, table):
    per_w = NLK // NWORKERS
    mesh = plsc.VectorSubcoreMesh(core_axis_name="c", subcore_axis_name="s")

    @functools.partial(
        pl.kernel,
        out_type=jax.ShapeDtypeStruct((NLK, D), jnp.float32),
        mesh=mesh,
        scratch_types=[
            pltpu.VMEM((CH,), jnp.int32),
            pltpu.VMEM((CH, D), jnp.float32),
            pltpu.SemaphoreType.DMA,
        ],
    )
    def gather_k(idx_hbm, tab_hbm, out_hbm, idx_v, rows_v, sem):
        wid = lax.axis_index("s") * 2 + lax.axis_index("c")
        base = wid * per_w

        def body(i, carry):
            off = base + i * CH
            pltpu.sync_copy(idx_hbm.at[pl.ds(off, CH)], idx_v)
            pltpu.async_copy(tab_hbm.at[idx_v], rows_v, sem).wait()
            pltpu.sync_copy(rows_v, out_hbm.at[pl.ds(off, CH)])
            return carry

        lax.fori_loop(0, per_w // CH, body, 0)

    return gather_k(idx_flat, table)


def kernel(struct, seq, token_to_decode, W_e, W_s, W1, W2, W_out):
    ca = struct[:, :, 1, :]
    ca_flat = ca.reshape(NL, 3)
    caT = jnp.transpose(ca, (0, 2, 1))
    seq_col = seq.reshape(NL, 1).astype(jnp.int32)
    ws_pad = jnp.zeros((32, D), jnp.float32).at[:21].set(W_s)
    wout_pad = jnp.zeros((D, D), jnp.float32).at[:, :21].set(W_out)

    idx, rbf, h = pl.pallas_call(
        _knn_body,
        grid=(NL // LBLK,),
        in_specs=[
            pl.BlockSpec((LBLK, 3), lambda b: (b, 0)),
            pl.BlockSpec((1, 3, L), lambda b: (b // (L // LBLK), 0, 0)),
            pl.BlockSpec((LBLK, 1), lambda b: (b, 0)),
            pl.BlockSpec((32, D), lambda b: (0, 0)),
        ],
        out_specs=[
            pl.BlockSpec((LBLK, K), lambda b: (b, 0)),
            pl.BlockSpec((LBLK, K, 16), lambda b: (b, 0, 0)),
            pl.BlockSpec((LBLK, D), lambda b: (b, 0)),
        ],
        out_shape=[
            jax.ShapeDtypeStruct((NL, K), jnp.int32),
            jax.ShapeDtypeStruct((NL, K, 16), jnp.float32),
            jax.ShapeDtypeStruct((NL, D), jnp.float32),
        ],
    )(ca_flat, caT, seq_col, ws_pad)
    idx_flat = idx.reshape(NLK)

    for l in range(3):
        w1ab = jnp.concatenate([W1[l, :D], W1[l, D : 2 * D]], axis=1)
        p, g = pl.pallas_call(
            _proj_body,
            grid=(NL // PBLK,),
            in_specs=[
                pl.BlockSpec((PBLK, D), lambda b: (b, 0)),
                pl.BlockSpec((D, 2 * D), lambda b: (0, 0)),
            ],
            out_specs=[
                pl.BlockSpec((PBLK, D), lambda b: (b, 0)),
                pl.BlockSpec((PBLK, D), lambda b: (b, 0)),
            ],
            out_shape=[
                jax.ShapeDtypeStruct((NL, D), jnp.float32),
                jax.ShapeDtypeStruct((NL, D), jnp.float32),
            ],
        )(h, w1ab)
        gj = _sc_gather(idx_flat, g)
        h = pl.pallas_call(
            _combine_body,
            grid=(NL // CBLK,),
            in_specs=[
                pl.BlockSpec((CBLK * K, D), lambda b: (b, 0)),
                pl.BlockSpec((CBLK, K, 16), lambda b: (b, 0, 0)),
                pl.BlockSpec((CBLK, D), lambda b: (b, 0)),
                pl.BlockSpec((CBLK, D), lambda b: (b, 0)),
                pl.BlockSpec((16, D), lambda b: (0, 0)),
                pl.BlockSpec((D, D), lambda b: (0, 0)),
                pl.BlockSpec((D, D), lambda b: (0, 0)),
            ],
            out_specs=pl.BlockSpec((CBLK, D), lambda b: (b, 0)),
            out_shape=jax.ShapeDtypeStruct((NL, D), jnp.float32),
        )(gj, rbf, p, h, W_e, W1[l, 2 * D :], W2[l])

    out = pl.pallas_call(
        _out_body,
        grid_spec=pltpu.PrefetchScalarGridSpec(
            num_scalar_prefetch=1,
            grid=(N,),
            in_specs=[
                pl.BlockSpec((1, L, D), lambda n, tok: (n, 0, 0)),
                pl.BlockSpec((D, D), lambda n, tok: (0, 0)),
            ],
            out_specs=pl.BlockSpec((1, 1, D), lambda n, tok: (n, 0, 0)),
        ),
        out_shape=jax.ShapeDtypeStruct((N, 1, D), jnp.float32),
    )(token_to_decode.astype(jnp.int32), h.reshape(N, L, D), wout_pad)
    return out.reshape(N, D)[:, :20]


# speedup vs baseline: 15.5654x; 1.4995x over previous
import functools

import jax
import jax.numpy as jnp
from jax import lax
from jax.experimental import pallas as pl
from jax.experimental.pallas import tpu as pltpu
from jax.experimental.pallas import tpu_sc as plsc

N, L, K, D = 8, 1024, 48, 128
NL = N * L
NLK = NL * K
LBLK = 256
PBLK = 512
CBLK = 128
NWORKERS = 32
CH = 128

_dot = functools.partial(
    jnp.dot, precision=lax.Precision.HIGHEST, preferred_element_type=jnp.float32
)


def _knn_body(ca_i_ref, caT_ref, seq_ref, ws_ref, idx_ref, rbf_ref, h0_ref):
    b = pl.program_id(0)
    n_off = (b // (L // LBLK)) * L
    d2 = jnp.zeros((LBLK, L), jnp.float32)
    for c in range(3):
        xi = ca_i_ref[:, c : c + 1]
        xj = caT_ref[0, c : c + 1, :]
        diff = xi - xj
        d2 = d2 + diff * diff
    jidx = lax.broadcasted_iota(jnp.int32, (LBLK, L), 1)
    keys = (lax.bitcast_convert_type(d2, jnp.int32) & jnp.int32(~1023)) | jidx
    idx_cols = []
    d_cols = []
    for _ in range(K):
        m = jnp.min(keys, axis=1, keepdims=True)
        idx_cols.append(m & 1023)
        d_cols.append(lax.bitcast_convert_type(m & jnp.int32(~1023), jnp.float32))
        keys = jnp.where(keys == m, jnp.int32(2**31 - 1), keys)
    idxs = jnp.concatenate(idx_cols, axis=1)
    dsel = jnp.sqrt(jnp.concatenate(d_cols, axis=1) + 1e-6)
    idx_ref[...] = idxs + n_off
    mu = 2.0 + (20.0 / 15.0) * lax.broadcasted_iota(
        jnp.int32, (LBLK, K, 16), 2
    ).astype(jnp.float32)
    z = (dsel[:, :, None] - mu) * (1.0 / 1.25)
    rbf_ref[...] = jnp.exp(-(z * z))
    s = seq_ref[...]
    oh = jnp.where(
        s == lax.broadcasted_iota(jnp.int32, (LBLK, 32), 1),
        jnp.float32(1.0),
        jnp.float32(0.0),
    )
    h0_ref[...] = _dot(oh, ws_ref[...])


def _proj_body(h_ref, w_ref, p_ref, g_ref):
    pg = _dot(h_ref[...], w_ref[...])
    p_ref[...] = pg[:, :D]
    g_ref[...] = pg[:, D:]


def _combine_body(gj_ref, rbf_ref, p_ref, h_ref, we_ref, w1c_ref, w2_ref, out_ref):
    u = _dot(we_ref[...], w1c_ref[...])
    r = rbf_ref[...].reshape(CBLK * K, 16)
    f = jnp.dot(
        r.astype(jnp.bfloat16),
        u.astype(jnp.bfloat16),
        preferred_element_type=jnp.float32,
    )
    a = (gj_ref[...].astype(jnp.float32) + f).reshape(CBLK, K, D) + p_ref[...][:, None, :]
    s = jnp.maximum(a, 0.0).sum(axis=1) * (1.0 / K)
    x = h_ref[...] + _dot(s, w2_ref[...])
    m = jnp.mean(x, axis=1, keepdims=True)
    xc = x - m
    var = jnp.mean(xc * xc, axis=1, keepdims=True)
    out_ref[...] = xc / jnp.sqrt(var + 1e-5)


def _out_body(tok_ref, h_ref, wout_ref, out_ref):
    n = pl.program_id(0)
    row = h_ref[0, pl.ds(tok_ref[n], 1), :]
    logits = _dot(row, wout_ref[...])
    lane = lax.broadcasted_iota(jnp.int32, (1, D), 1)
    logits = jnp.where(lane < 20, logits, jnp.float32(-3.0e38))
    mx = jnp.max(logits, axis=1, keepdims=True)
    e = jnp.exp(logits - mx)
    out_ref[0] = e / jnp.sum(e, axis=1, keepdims=True)


def _sc_gather(idx_flat, table):
    per_w = NLK // NWORKERS
    mesh = plsc.VectorSubcoreMesh(core_axis_name="c", subcore_axis_name="s")

    @functools.partial(
        pl.kernel,
        out_type=jax.ShapeDtypeStruct((NLK, D), jnp.float32),
        mesh=mesh,
        scratch_types=[
            pltpu.VMEM((2, CH), jnp.int32),
            pltpu.VMEM((2, CH, D), jnp.float32),
            pltpu.SemaphoreType.DMA,
            pltpu.SemaphoreType.DMA,
            pltpu.SemaphoreType.DMA,
            pltpu.SemaphoreType.DMA,
            pltpu.SemaphoreType.DMA,
            pltpu.SemaphoreType.DMA,
        ],
    )
    def gather_k(idx_hbm, tab_hbm, out_hbm, idx_v, rows_v, semi0, semi1,
                 semg0, semg1, semo0, semo1):
        wid = lax.axis_index("s") * 2 + lax.axis_index("c")
        base = wid * per_w
        semi = (semi0, semi1)
        semg = (semg0, semg1)
        semo = (semo0, semo1)
        nch = per_w // CH

        def idx_cp(j, b):
            return pltpu.make_async_copy(
                idx_hbm.at[pl.ds(base + j * CH, CH)], idx_v.at[b], semi[b]
            )

        def gat_cp(b):
            return pltpu.make_async_copy(
                tab_hbm.at[idx_v.at[b]], rows_v.at[b], semg[b]
            )

        def out_cp(j, b):
            return pltpu.make_async_copy(
                rows_v.at[b], out_hbm.at[pl.ds(base + j * CH, CH)], semo[b]
            )

        idx_cp(0, 0).start()

        def body(i, carry):
            j0 = 2 * i
            idx_cp(j0, 0).wait()
            idx_cp(j0 + 1, 1).start()

            @pl.when(i > 0)
            def _():
                out_cp(j0 - 2, 0).wait()

            gat_cp(0).start()
            gat_cp(0).wait()
            out_cp(j0, 0).start()

            idx_cp(j0 + 1, 1).wait()

            @pl.when(i < nch // 2 - 1)
            def _():
                idx_cp(j0 + 2, 0).start()

            @pl.when(i > 0)
            def _():
                out_cp(j0 - 1, 1).wait()

            gat_cp(1).start()
            gat_cp(1).wait()
            out_cp(j0 + 1, 1).start()
            return carry

        lax.fori_loop(0, nch // 2, body, 0)
        out_cp(nch - 2, 0).wait()
        out_cp(nch - 1, 1).wait()

    return gather_k(idx_flat, table)


def kernel(struct, seq, token_to_decode, W_e, W_s, W1, W2, W_out):
    ca = struct[:, :, 1, :]
    ca_flat = ca.reshape(NL, 3)
    caT = jnp.transpose(ca, (0, 2, 1))
    seq_col = seq.reshape(NL, 1).astype(jnp.int32)
    ws_pad = jnp.zeros((32, D), jnp.float32).at[:21].set(W_s)
    wout_pad = jnp.zeros((D, D), jnp.float32).at[:, :21].set(W_out)

    idx, rbf, h = pl.pallas_call(
        _knn_body,
        grid=(NL // LBLK,),
        in_specs=[
            pl.BlockSpec((LBLK, 3), lambda b: (b, 0)),
            pl.BlockSpec((1, 3, L), lambda b: (b // (L // LBLK), 0, 0)),
            pl.BlockSpec((LBLK, 1), lambda b: (b, 0)),
            pl.BlockSpec((32, D), lambda b: (0, 0)),
        ],
        out_specs=[
            pl.BlockSpec((LBLK, K), lambda b: (b, 0)),
            pl.BlockSpec((LBLK, K, 16), lambda b: (b, 0, 0)),
            pl.BlockSpec((LBLK, D), lambda b: (b, 0)),
        ],
        out_shape=[
            jax.ShapeDtypeStruct((NL, K), jnp.int32),
            jax.ShapeDtypeStruct((NL, K, 16), jnp.float32),
            jax.ShapeDtypeStruct((NL, D), jnp.float32),
        ],
    )(ca_flat, caT, seq_col, ws_pad)
    idx_flat = idx.reshape(NLK)

    for l in range(3):
        w1ab = jnp.concatenate([W1[l, :D], W1[l, D : 2 * D]], axis=1)
        p, g = pl.pallas_call(
            _proj_body,
            grid=(NL // PBLK,),
            in_specs=[
                pl.BlockSpec((PBLK, D), lambda b: (b, 0)),
                pl.BlockSpec((D, 2 * D), lambda b: (0, 0)),
            ],
            out_specs=[
                pl.BlockSpec((PBLK, D), lambda b: (b, 0)),
                pl.BlockSpec((PBLK, D), lambda b: (b, 0)),
            ],
            out_shape=[
                jax.ShapeDtypeStruct((NL, D), jnp.float32),
                jax.ShapeDtypeStruct((NL, D), jnp.float32),
            ],
        )(h, w1ab)
        gj = _sc_gather(idx_flat, g)
        h = pl.pallas_call(
            _combine_body,
            grid=(NL // CBLK,),
            in_specs=[
                pl.BlockSpec((CBLK * K, D), lambda b: (b, 0)),
                pl.BlockSpec((CBLK, K, 16), lambda b: (b, 0, 0)),
                pl.BlockSpec((CBLK, D), lambda b: (b, 0)),
                pl.BlockSpec((CBLK, D), lambda b: (b, 0)),
                pl.BlockSpec((16, D), lambda b: (0, 0)),
                pl.BlockSpec((D, D), lambda b: (0, 0)),
                pl.BlockSpec((D, D), lambda b: (0, 0)),
            ],
            out_specs=pl.BlockSpec((CBLK, D), lambda b: (b, 0)),
            out_shape=jax.ShapeDtypeStruct((NL, D), jnp.float32),
        )(gj, rbf, p, h, W_e, W1[l, 2 * D :], W2[l])

    out = pl.pallas_call(
        _out_body,
        grid_spec=pltpu.PrefetchScalarGridSpec(
            num_scalar_prefetch=1,
            grid=(N,),
            in_specs=[
                pl.BlockSpec((1, L, D), lambda n, tok: (n, 0, 0)),
                pl.BlockSpec((D, D), lambda n, tok: (0, 0)),
            ],
            out_specs=pl.BlockSpec((1, 1, D), lambda n, tok: (n, 0, 0)),
        ),
        out_shape=jax.ShapeDtypeStruct((N, 1, D), jnp.float32),
    )(token_to_decode.astype(jnp.int32), h.reshape(N, L, D), wout_pad)
    return out.reshape(N, D)[:, :20]


# speedup vs baseline: 15.7500x; 1.0119x over previous
import functools

import jax
import jax.numpy as jnp
from jax import lax
from jax.experimental import pallas as pl
from jax.experimental.pallas import tpu as pltpu
from jax.experimental.pallas import tpu_sc as plsc

N, L, K, D = 8, 1024, 48, 128
NL = N * L
NLK = NL * K
LBLK = 256
PBLK = 512
CBLK = 128
NWORKERS = 32
CH = 128

_dot = functools.partial(
    jnp.dot, precision=lax.Precision.HIGHEST, preferred_element_type=jnp.float32
)


def _knn_body(ca_i_ref, caT_ref, seq_ref, ws_ref, wab_ref,
              idx_ref, rbf_ref, h0_ref, p_ref, g_ref):
    b = pl.program_id(0)
    n_off = (b // (L // LBLK)) * L
    d2 = jnp.zeros((LBLK, L), jnp.float32)
    for c in range(3):
        xi = ca_i_ref[:, c : c + 1]
        xj = caT_ref[0, c : c + 1, :]
        diff = xi - xj
        d2 = d2 + diff * diff
    jidx = lax.broadcasted_iota(jnp.int32, (LBLK, L), 1)
    keys = (lax.bitcast_convert_type(d2, jnp.int32) & jnp.int32(~1023)) | jidx
    idx_cols = []
    d_cols = []
    for _ in range(K):
        m = jnp.min(keys, axis=1, keepdims=True)
        idx_cols.append(m & 1023)
        d_cols.append(lax.bitcast_convert_type(m & jnp.int32(~1023), jnp.float32))
        keys = jnp.where(keys == m, jnp.int32(2**31 - 1), keys)
    idxs = jnp.concatenate(idx_cols, axis=1)
    dsel = jnp.sqrt(jnp.concatenate(d_cols, axis=1) + 1e-6)
    idx_ref[...] = idxs + n_off
    mu = 2.0 + (20.0 / 15.0) * lax.broadcasted_iota(
        jnp.int32, (LBLK, K, 16), 2
    ).astype(jnp.float32)
    z = (dsel[:, :, None] - mu) * (1.0 / 1.25)
    rbf_ref[...] = jnp.exp(-(z * z))
    s = seq_ref[...]
    oh = jnp.where(
        s == lax.broadcasted_iota(jnp.int32, (LBLK, 32), 1),
        jnp.float32(1.0),
        jnp.float32(0.0),
    )
    h0 = _dot(oh, ws_ref[...])
    h0_ref[...] = h0
    wab = wab_ref[0]
    p_ref[...] = _dot(h0, wab[:D])
    g_ref[...] = _dot(h0, wab[D:])


def _combine_body(has_next, gj_ref, rbf_ref, p_ref, h_ref, we_ref, w1c_ref,
                  w2_ref, *rest):
    if has_next:
        wabn_ref, out_ref, pn_ref, gn_ref = rest
    else:
        (out_ref,) = rest
    u = _dot(we_ref[...], w1c_ref[0])
    r = rbf_ref[...].reshape(CBLK * K, 16)
    f = jnp.dot(
        r.astype(jnp.bfloat16),
        u.astype(jnp.bfloat16),
        preferred_element_type=jnp.float32,
    )
    a = (gj_ref[...].astype(jnp.float32) + f).reshape(CBLK, K, D) + p_ref[...][:, None, :]
    s = jnp.maximum(a, 0.0).sum(axis=1) * (1.0 / K)
    x = h_ref[...] + _dot(s, w2_ref[0])
    m = jnp.mean(x, axis=1, keepdims=True)
    xc = x - m
    var = jnp.mean(xc * xc, axis=1, keepdims=True)
    hn = xc / jnp.sqrt(var + 1e-5)
    out_ref[...] = hn
    if has_next:
        wab = wabn_ref[0]
        pn_ref[...] = _dot(hn, wab[:D])
        gn_ref[...] = _dot(hn, wab[D:])


def _out_body(tok_ref, h_ref, wout_ref, out_ref):
    n = pl.program_id(0)
    row = h_ref[0, pl.ds(tok_ref[n], 1), :]
    logits = _dot(row, wout_ref[...])
    lane = lax.broadcasted_iota(jnp.int32, (1, D), 1)
    logits = jnp.where(lane < 20, logits, jnp.float32(-3.0e38))
    mx = jnp.max(logits, axis=1, keepdims=True)
    e = jnp.exp(logits - mx)
    out_ref[0] = e / jnp.sum(e, axis=1, keepdims=True)


def _sc_gather(idx_flat, table):
    per_w = NLK // NWORKERS
    mesh = plsc.VectorSubcoreMesh(core_axis_name="c", subcore_axis_name="s")

    @functools.partial(
        pl.kernel,
        out_type=jax.ShapeDtypeStruct((NLK, D), jnp.float32),
        mesh=mesh,
        scratch_types=[
            pltpu.VMEM((2, CH), jnp.int32),
            pltpu.VMEM((2, CH, D), jnp.float32),
            pltpu.SemaphoreType.DMA,
            pltpu.SemaphoreType.DMA,
            pltpu.SemaphoreType.DMA,
            pltpu.SemaphoreType.DMA,
            pltpu.SemaphoreType.DMA,
            pltpu.SemaphoreType.DMA,
        ],
    )
    def gather_k(idx_hbm, tab_hbm, out_hbm, idx_v, rows_v, semi0, semi1,
                 semg0, semg1, semo0, semo1):
        wid = lax.axis_index("s") * 2 + lax.axis_index("c")
        base = wid * per_w
        semi = (semi0, semi1)
        semg = (semg0, semg1)
        semo = (semo0, semo1)
        nch = per_w // CH

        def idx_cp(j, b):
            return pltpu.make_async_copy(
                idx_hbm.at[pl.ds(base + j * CH, CH)], idx_v.at[b], semi[b]
            )

        def gat_cp(b):
            return pltpu.make_async_copy(
                tab_hbm.at[idx_v.at[b]], rows_v.at[b], semg[b]
            )

        def out_cp(j, b):
            return pltpu.make_async_copy(
                rows_v.at[b], out_hbm.at[pl.ds(base + j * CH, CH)], semo[b]
            )

        idx_cp(0, 0).start()

        def body(i, carry):
            j0 = 2 * i
            idx_cp(j0, 0).wait()
            idx_cp(j0 + 1, 1).start()

            @pl.when(i > 0)
            def _():
                out_cp(j0 - 2, 0).wait()

            gat_cp(0).start()
            gat_cp(0).wait()
            out_cp(j0, 0).start()

            idx_cp(j0 + 1, 1).wait()

            @pl.when(i < nch // 2 - 1)
            def _():
                idx_cp(j0 + 2, 0).start()

            @pl.when(i > 0)
            def _():
                out_cp(j0 - 1, 1).wait()

            gat_cp(1).start()
            gat_cp(1).wait()
            out_cp(j0 + 1, 1).start()
            return carry

        lax.fori_loop(0, nch // 2, body, 0)
        out_cp(nch - 2, 0).wait()
        out_cp(nch - 1, 1).wait()

    return gather_k(idx_flat, table)


def kernel(struct, seq, token_to_decode, W_e, W_s, W1, W2, W_out):
    ca = struct[:, :, 1, :]
    ca_flat = ca.reshape(NL, 3)
    caT = jnp.transpose(ca, (0, 2, 1))
    seq_col = seq.reshape(NL, 1).astype(jnp.int32)
    ws_pad = jnp.zeros((32, D), jnp.float32).at[:21].set(W_s)
    wout_pad = jnp.zeros((D, D), jnp.float32).at[:, :21].set(W_out)

    idx, rbf, h, p, g = pl.pallas_call(
        _knn_body,
        grid=(NL // LBLK,),
        in_specs=[
            pl.BlockSpec((LBLK, 3), lambda b: (b, 0)),
            pl.BlockSpec((1, 3, L), lambda b: (b // (L // LBLK), 0, 0)),
            pl.BlockSpec((LBLK, 1), lambda b: (b, 0)),
            pl.BlockSpec((32, D), lambda b: (0, 0)),
            pl.BlockSpec((1, 2 * D, D), lambda b: (0, 0, 0)),
        ],
        out_specs=[
            pl.BlockSpec((LBLK, K), lambda b: (b, 0)),
            pl.BlockSpec((LBLK, K, 16), lambda b: (b, 0, 0)),
            pl.BlockSpec((LBLK, D), lambda b: (b, 0)),
            pl.BlockSpec((LBLK, D), lambda b: (b, 0)),
            pl.BlockSpec((LBLK, D), lambda b: (b, 0)),
        ],
        out_shape=[
            jax.ShapeDtypeStruct((NL, K), jnp.int32),
            jax.ShapeDtypeStruct((NL, K, 16), jnp.float32),
            jax.ShapeDtypeStruct((NL, D), jnp.float32),
            jax.ShapeDtypeStruct((NL, D), jnp.float32),
            jax.ShapeDtypeStruct((NL, D), jnp.float32),
        ],
    )(ca_flat, caT, seq_col, ws_pad, W1)
    idx_flat = idx.reshape(NLK)

    for l in range(3):
        gj = _sc_gather(idx_flat, g)
        has_next = l < 2
        in_specs = [
            pl.BlockSpec((CBLK * K, D), lambda b: (b, 0)),
            pl.BlockSpec((CBLK, K, 16), lambda b: (b, 0, 0)),
            pl.BlockSpec((CBLK, D), lambda b: (b, 0)),
            pl.BlockSpec((CBLK, D), lambda b: (b, 0)),
            pl.BlockSpec((16, D), lambda b: (0, 0)),
            pl.BlockSpec((1, D, D), lambda b, _l=l: (_l, 2, 0)),
            pl.BlockSpec((1, D, D), lambda b, _l=l: (_l, 0, 0)),
        ]
        operands = [gj, rbf, p, h, W_e, W1, W2]
        out_specs = [pl.BlockSpec((CBLK, D), lambda b: (b, 0))]
        out_shape = [jax.ShapeDtypeStruct((NL, D), jnp.float32)]
        if has_next:
            in_specs.append(
                pl.BlockSpec((1, 2 * D, D), lambda b, _l=l: (_l + 1, 0, 0))
            )
            operands.append(W1)
            out_specs += [pl.BlockSpec((CBLK, D), lambda b: (b, 0))] * 2
            out_shape += [jax.ShapeDtypeStruct((NL, D), jnp.float32)] * 2
        res = pl.pallas_call(
            functools.partial(_combine_body, has_next),
            grid=(NL // CBLK,),
            in_specs=in_specs,
            out_specs=out_specs,
            out_shape=out_shape,
        )(*operands)
        if has_next:
            h, p, g = res
        else:
            (h,) = res

    out = pl.pallas_call(
        _out_body,
        grid_spec=pltpu.PrefetchScalarGridSpec(
            num_scalar_prefetch=1,
            grid=(N,),
            in_specs=[
                pl.BlockSpec((1, L, D), lambda n, tok: (n, 0, 0)),
                pl.BlockSpec((D, D), lambda n, tok: (0, 0)),
            ],
            out_specs=pl.BlockSpec((1, 1, D), lambda n, tok: (n, 0, 0)),
        ),
        out_shape=jax.ShapeDtypeStruct((N, 1, D), jnp.float32),
    )(token_to_decode.astype(jnp.int32), h.reshape(N, L, D), wout_pad)
    return out.reshape(N, D)[:, :20]


# speedup vs baseline: 16.2371x; 1.0309x over previous
import functools

import jax
import jax.numpy as jnp
from jax import lax
from jax.experimental import pallas as pl
from jax.experimental.pallas import tpu as pltpu
from jax.experimental.pallas import tpu_sc as plsc

N, L, K, D = 8, 1024, 48, 128
NL = N * L
NLK = NL * K
LBLK = 256
PBLK = 512
CBLK = 128
NWORKERS = 32
CH = 128

_dot = functools.partial(
    jnp.dot, precision=lax.Precision.HIGHEST, preferred_element_type=jnp.float32
)


def _knn_body(ca_i_ref, caT_ref, seq_ref, ws_ref, wab_ref,
              idx_ref, rbf_ref, h0_ref, p_ref, g_ref):
    b = pl.program_id(0)
    n_off = (b // (L // LBLK)) * L
    d2 = jnp.zeros((LBLK, L), jnp.float32)
    for c in range(3):
        xi = ca_i_ref[:, c : c + 1]
        xj = caT_ref[0, c : c + 1, :]
        diff = xi - xj
        d2 = d2 + diff * diff
    jidx = lax.broadcasted_iota(jnp.int32, (LBLK, L), 1)
    keys = (lax.bitcast_convert_type(d2, jnp.int32) & jnp.int32(~1023)) | jidx
    idx_cols = []
    d_cols = []
    for _ in range(K):
        m = jnp.min(keys, axis=1, keepdims=True)
        idx_cols.append(m & 1023)
        d_cols.append(lax.bitcast_convert_type(m & jnp.int32(~1023), jnp.float32))
        keys = jnp.where(keys == m, jnp.int32(2**31 - 1), keys)
    idxs = jnp.concatenate(idx_cols, axis=1)
    dsel = jnp.sqrt(jnp.concatenate(d_cols, axis=1) + 1e-6)
    idx_ref[...] = idxs + n_off
    mu = 2.0 + (20.0 / 15.0) * lax.broadcasted_iota(
        jnp.int32, (LBLK, K, 16), 2
    ).astype(jnp.float32)
    z = (dsel[:, :, None] - mu) * (1.0 / 1.25)
    rbf_ref[...] = jnp.exp(-(z * z))
    s = seq_ref[...]
    oh = jnp.where(
        s == lax.broadcasted_iota(jnp.int32, (LBLK, 32), 1),
        jnp.float32(1.0),
        jnp.float32(0.0),
    )
    h0 = _dot(oh, ws_ref[...])
    h0_ref[...] = h0
    wab = wab_ref[0]
    p_ref[...] = _dot(h0, wab[:D])
    g_ref[...] = _dot(h0, wab[D:])


def _combine_body(has_next, gj_ref, rbf_ref, p_ref, h_ref, we_ref, w1c_ref,
                  w2_ref, *rest):
    if has_next:
        wabn_ref, out_ref, pn_ref, gn_ref = rest
    else:
        (out_ref,) = rest
    u = _dot(we_ref[...], w1c_ref[0])
    r = rbf_ref[...].reshape(CBLK * K, 16)
    f = jnp.dot(
        r.astype(jnp.bfloat16),
        u.astype(jnp.bfloat16),
        preferred_element_type=jnp.float32,
    )
    a = (gj_ref[...].astype(jnp.float32) + f).reshape(CBLK, K, D) + p_ref[...][:, None, :]
    s = jnp.maximum(a, 0.0).sum(axis=1) * (1.0 / K)
    x = h_ref[...] + _dot(s, w2_ref[0])
    m = jnp.mean(x, axis=1, keepdims=True)
    xc = x - m
    var = jnp.mean(xc * xc, axis=1, keepdims=True)
    hn = xc / jnp.sqrt(var + 1e-5)
    out_ref[...] = hn
    if has_next:
        wab = wabn_ref[0]
        pn_ref[...] = _dot(hn, wab[:D])
        gn_ref[...] = _dot(hn, wab[D:])


def _out_body(tok_ref, h_ref, wout_ref, out_ref):
    n = pl.program_id(0)
    row = h_ref[0, pl.ds(tok_ref[n], 1), :]
    logits = _dot(row, wout_ref[...])
    lane = lax.broadcasted_iota(jnp.int32, (1, D), 1)
    logits = jnp.where(lane < 20, logits, jnp.float32(-3.0e38))
    mx = jnp.max(logits, axis=1, keepdims=True)
    e = jnp.exp(logits - mx)
    out_ref[0] = e / jnp.sum(e, axis=1, keepdims=True)


def _sc_gather(idx_flat, table, half):
    nrows = NLK // 2
    per_w = nrows // NWORKERS
    mesh = plsc.VectorSubcoreMesh(core_axis_name="c", subcore_axis_name="s")

    @functools.partial(
        pl.kernel,
        out_type=jax.ShapeDtypeStruct((nrows, D), jnp.float32),
        mesh=mesh,
        scratch_types=[
            pltpu.VMEM((2, CH), jnp.int32),
            pltpu.VMEM((2, CH, D), jnp.float32),
            pltpu.SemaphoreType.DMA,
            pltpu.SemaphoreType.DMA,
            pltpu.SemaphoreType.DMA,
            pltpu.SemaphoreType.DMA,
            pltpu.SemaphoreType.DMA,
            pltpu.SemaphoreType.DMA,
        ],
    )
    def gather_k(idx_hbm, tab_hbm, out_hbm, idx_v, rows_v, semi0, semi1,
                 semg0, semg1, semo0, semo1):
        wid = lax.axis_index("s") * 2 + lax.axis_index("c")
        base = wid * per_w
        ibase = half * nrows + base
        semi = (semi0, semi1)
        semg = (semg0, semg1)
        semo = (semo0, semo1)
        nch = per_w // CH

        def idx_cp(j, b):
            return pltpu.make_async_copy(
                idx_hbm.at[pl.ds(ibase + j * CH, CH)], idx_v.at[b], semi[b]
            )

        def gat_cp(b):
            return pltpu.make_async_copy(
                tab_hbm.at[idx_v.at[b]], rows_v.at[b], semg[b]
            )

        def out_cp(j, b):
            return pltpu.make_async_copy(
                rows_v.at[b], out_hbm.at[pl.ds(base + j * CH, CH)], semo[b]
            )

        idx_cp(0, 0).start()

        def body(i, carry):
            j0 = 2 * i
            idx_cp(j0, 0).wait()
            idx_cp(j0 + 1, 1).start()

            @pl.when(i > 0)
            def _():
                out_cp(j0 - 2, 0).wait()

            gat_cp(0).start()
            gat_cp(0).wait()
            out_cp(j0, 0).start()

            idx_cp(j0 + 1, 1).wait()

            @pl.when(i < nch // 2 - 1)
            def _():
                idx_cp(j0 + 2, 0).start()

            @pl.when(i > 0)
            def _():
                out_cp(j0 - 1, 1).wait()

            gat_cp(1).start()
            gat_cp(1).wait()
            out_cp(j0 + 1, 1).start()
            return carry

        lax.fori_loop(0, nch // 2, body, 0)
        out_cp(nch - 2, 0).wait()
        out_cp(nch - 1, 1).wait()

    return gather_k(idx_flat, table)


def kernel(struct, seq, token_to_decode, W_e, W_s, W1, W2, W_out):
    ca = struct[:, :, 1, :]
    ca_flat = ca.reshape(NL, 3)
    caT = jnp.transpose(ca, (0, 2, 1))
    seq_col = seq.reshape(NL, 1).astype(jnp.int32)
    ws_pad = jnp.zeros((32, D), jnp.float32).at[:21].set(W_s)
    wout_pad = jnp.zeros((D, D), jnp.float32).at[:, :21].set(W_out)

    idx, rbf, h, p, g = pl.pallas_call(
        _knn_body,
        grid=(NL // LBLK,),
        in_specs=[
            pl.BlockSpec((LBLK, 3), lambda b: (b, 0)),
            pl.BlockSpec((1, 3, L), lambda b: (b // (L // LBLK), 0, 0)),
            pl.BlockSpec((LBLK, 1), lambda b: (b, 0)),
            pl.BlockSpec((32, D), lambda b: (0, 0)),
            pl.BlockSpec((1, 2 * D, D), lambda b: (0, 0, 0)),
        ],
        out_specs=[
            pl.BlockSpec((LBLK, K), lambda b: (b, 0)),
            pl.BlockSpec((LBLK, K, 16), lambda b: (b, 0, 0)),
            pl.BlockSpec((LBLK, D), lambda b: (b, 0)),
            pl.BlockSpec((LBLK, D), lambda b: (b, 0)),
            pl.BlockSpec((LBLK, D), lambda b: (b, 0)),
        ],
        out_shape=[
            jax.ShapeDtypeStruct((NL, K), jnp.int32),
            jax.ShapeDtypeStruct((NL, K, 16), jnp.float32),
            jax.ShapeDtypeStruct((NL, D), jnp.float32),
            jax.ShapeDtypeStruct((NL, D), jnp.float32),
            jax.ShapeDtypeStruct((NL, D), jnp.float32),
        ],
    )(ca_flat, caT, seq_col, ws_pad, W1)
    idx_flat = idx.reshape(NLK)

    NLH = NL // 2
    h = (h, h)
    p = (p, p)

    def combine_half(l, half, gj, ph, hh, row_off):
        has_next = l < 2
        rbf_off = half * (NLH // CBLK)
        in_specs = [
            pl.BlockSpec((CBLK * K, D), lambda b: (b, 0)),
            pl.BlockSpec((CBLK, K, 16), lambda b, _o=rbf_off: (b + _o, 0, 0)),
            pl.BlockSpec((CBLK, D), lambda b, _o=row_off: (b + _o, 0)),
            pl.BlockSpec((CBLK, D), lambda b, _o=row_off: (b + _o, 0)),
            pl.BlockSpec((16, D), lambda b: (0, 0)),
            pl.BlockSpec((1, D, D), lambda b, _l=l: (_l, 2, 0)),
            pl.BlockSpec((1, D, D), lambda b, _l=l: (_l, 0, 0)),
        ]
        operands = [gj, rbf, ph, hh, W_e, W1, W2]
        out_specs = [pl.BlockSpec((CBLK, D), lambda b: (b, 0))]
        out_shape = [jax.ShapeDtypeStruct((NLH, D), jnp.float32)]
        if has_next:
            in_specs.append(
                pl.BlockSpec((1, 2 * D, D), lambda b, _l=l: (_l + 1, 0, 0))
            )
            operands.append(W1)
            out_specs += [pl.BlockSpec((CBLK, D), lambda b: (b, 0))] * 2
            out_shape += [jax.ShapeDtypeStruct((NLH, D), jnp.float32)] * 2
        res = pl.pallas_call(
            functools.partial(_combine_body, has_next),
            grid=(NLH // CBLK,),
            in_specs=in_specs,
            out_specs=out_specs,
            out_shape=out_shape,
        )(*operands)
        return res if has_next else (res[0], None, None)

    for l in range(3):
        gj0 = _sc_gather(idx_flat, g, 0)
        gj1 = _sc_gather(idx_flat, g, 1)
        off0 = 0
        off1 = NLH // CBLK if l == 0 else 0
        h0, p0, g0 = combine_half(l, 0, gj0, p[0], h[0], off0)
        h1, p1, g1 = combine_half(l, 1, gj1, p[1], h[1], off1)
        h, p = (h0, h1), (p0, p1)
        if l < 2:
            g = jnp.concatenate([g0, g1], axis=0)
    h = jnp.concatenate([h[0], h[1]], axis=0)

    out = pl.pallas_call(
        _out_body,
        grid_spec=pltpu.PrefetchScalarGridSpec(
            num_scalar_prefetch=1,
            grid=(N,),
            in_specs=[
                pl.BlockSpec((1, L, D), lambda n, tok: (n, 0, 0)),
                pl.BlockSpec((D, D), lambda n, tok: (0, 0)),
            ],
            out_specs=pl.BlockSpec((1, 1, D), lambda n, tok: (n, 0, 0)),
        ),
        out_shape=jax.ShapeDtypeStruct((N, 1, D), jnp.float32),
    )(token_to_decode.astype(jnp.int32), h.reshape(N, L, D), wout_pad)
    return out.reshape(N, D)[:, :20]
